# baseline (speedup 1.0000x reference)
import functools

import jax
import jax.numpy as jnp
from jax import lax
from jax.experimental import pallas as pl
from jax.experimental.pallas import tpu as pltpu
from jax.experimental.pallas import tpu_sc as plsc

VOCAB = 100000
EMB = 128
B = 4096
L = 50

_NC = 2
_NS = 16
_NW = _NC * _NS
_BW = B // _NW


def _make_kernel():
    mesh = plsc.VectorSubcoreMesh(core_axis_name="c", subcore_axis_name="s")

    @functools.partial(
        pl.kernel,
        mesh=mesh,
        out_type=jax.ShapeDtypeStruct((B, L, EMB), jnp.float32),
        scratch_types=[
            pltpu.VMEM((_BW, L), jnp.int32),
            pltpu.VMEM((L, EMB), jnp.float32),
            pltpu.VMEM((L, EMB), jnp.float32),
            pltpu.SemaphoreType.DMA,
            pltpu.SemaphoreType.DMA,
        ],
    )
    def k(idx_hbm, table_hbm, out_hbm, idx_v, rows0, rows1, sem0, sem1):
        wid = lax.axis_index("s") * _NC + lax.axis_index("c")
        base = wid * _BW
        pltpu.sync_copy(idx_hbm.at[wid], idx_v)

        def start_gather(j, rows, sem):
            pltpu.async_copy(table_hbm.at[idx_v.at[j]], rows, sem)

        def wait_gather(j, rows, sem):
            pltpu.make_async_copy(table_hbm.at[idx_v.at[j]], rows, sem).wait()

        def write(j, rows):
            pltpu.sync_copy(rows, out_hbm.at[base + j])

        start_gather(0, rows0, sem0)

        def body(g, _):
            j = 2 * g
            wait_gather(j, rows0, sem0)
            start_gather(j + 1, rows1, sem1)
            write(j, rows0)
            wait_gather(j + 1, rows1, sem1)
            start_gather(j + 2, rows0, sem0)
            write(j + 1, rows1)
            return 0

        lax.fori_loop(0, _BW // 2 - 1, body, 0)

        j = _BW - 2
        wait_gather(j, rows0, sem0)
        start_gather(j + 1, rows1, sem1)
        write(j, rows0)
        wait_gather(j + 1, rows1, sem1)
        write(j + 1, rows1)

    return k


_gather_kernel = _make_kernel()


@jax.jit
def kernel(x, weight):
    idx = x.astype(jnp.int32).reshape(_NW, _BW, L)
    return _gather_kernel(idx, weight)
